# baseline (speedup 1.0000x reference)
import functools

import jax
import jax.numpy as jnp
from jax import lax
from jax.experimental import pallas as pl
from jax.experimental.pallas import tpu as pltpu
from jax.experimental.pallas import tpu_sc as plsc

N_NODES = 10000
D = 128
NC, NS = 1, 16
NW = NC * NS
CHUNK = 128
N_ACC = 10112
ROWS_PER_TILE = N_ACC // NS
DEG_W = 16
SLAB = 8


def _sc_agg_body(src_hbm, dst_hbm, feat_hbm, sum_out, deg_out,
                 src_v, dst_v, rows_a, rows_b, ones_v, zdeg_v,
                 accum, degacc, gsem_a, gsem_b, ssem_a, ssem_b):
    c = lax.axis_index("c")
    s = lax.axis_index("s")
    w = s * NC + c
    cpw = src_hbm.shape[0] // NW

    zero16 = jnp.zeros((16,), jnp.float32)
    one16 = jnp.ones((16,), jnp.float32)

    def z_rows(i, _):
        rows_a[i // 8, pl.ds((i % 8) * 16, 16)] = zero16
        return 0
    lax.fori_loop(0, CHUNK * 8, z_rows, 0)

    def z_deg(i, _):
        zdeg_v[i, pl.ds(0, 16)] = zero16
        return 0
    lax.fori_loop(0, CHUNK, z_deg, 0)

    def o_rows(i, _):
        ones_v[i, pl.ds(0, 16)] = one16
        return 0
    lax.fori_loop(0, CHUNK, o_rows, 0)

    base = s * ROWS_PER_TILE
    for k in range(ROWS_PER_TILE // CHUNK):
        pltpu.sync_copy(rows_a, accum.at[pl.ds(base + k * CHUNK, CHUNK)])
        pltpu.sync_copy(zdeg_v, degacc.at[pl.ds(base + k * CHUNK, CHUNK)])
    rem = ROWS_PER_TILE % CHUNK
    if rem:
        rbase = base + (ROWS_PER_TILE // CHUNK) * CHUNK
        pltpu.sync_copy(rows_a.at[pl.ds(0, rem)], accum.at[pl.ds(rbase, rem)])
        pltpu.sync_copy(zdeg_v.at[pl.ds(0, rem)], degacc.at[pl.ds(rbase, rem)])
    plsc.subcore_barrier()

    bufs = (rows_a, rows_b)
    gsems = (gsem_a, gsem_b)
    ssems = (ssem_a, ssem_b)

    def outer(t, _):
        cbase = w * cpw + t * SLAB
        pltpu.sync_copy(src_hbm.at[pl.ds(cbase, SLAB)], src_v)
        pltpu.sync_copy(dst_hbm.at[pl.ds(cbase, SLAB)], dst_v)
        gather = {0: pltpu.async_copy(feat_hbm.at[src_v.at[0]], bufs[0], gsems[0])}
        pend = {0: [], 1: []}
        for j in range(SLAB):
            b = j % 2
            nb = 1 - b
            if j + 1 < SLAB:
                for h in pend[nb]:
                    h.wait()
                pend[nb] = []
                gather[nb] = pltpu.async_copy(
                    feat_hbm.at[src_v.at[j + 1]], bufs[nb], gsems[nb])
            gather[b].wait()
            pend[b] = [
                pltpu.async_copy(bufs[b], accum.at[dst_v.at[j]], ssems[b], add=True),
                pltpu.async_copy(ones_v, degacc.at[dst_v.at[j]], ssems[b], add=True),
            ]
        for k in (0, 1):
            for h in pend[k]:
                h.wait()
        return 0
    lax.fori_loop(0, cpw // SLAB, outer, 0)

    plsc.subcore_barrier()

    pltpu.sync_copy(accum.at[pl.ds(base, ROWS_PER_TILE)],
                    sum_out.at[c, pl.ds(base, ROWS_PER_TILE)])
    pltpu.sync_copy(degacc.at[pl.ds(base, ROWS_PER_TILE)],
                    deg_out.at[c, pl.ds(base, ROWS_PER_TILE)])


def _make_sc_agg(n_chunks):
    return pl.kernel(
        _sc_agg_body,
        out_type=[
            jax.ShapeDtypeStruct((NC, N_ACC, D), jnp.float32),
            jax.ShapeDtypeStruct((NC, N_ACC, DEG_W), jnp.float32),
        ],
        mesh=plsc.VectorSubcoreMesh(core_axis_name="c", subcore_axis_name="s",
                                    num_cores=NC),
        scratch_types=[
            pltpu.VMEM((SLAB, CHUNK), jnp.int32),
            pltpu.VMEM((SLAB, CHUNK), jnp.int32),
            pltpu.VMEM((CHUNK, D), jnp.float32),
            pltpu.VMEM((CHUNK, D), jnp.float32),
            pltpu.VMEM((CHUNK, DEG_W), jnp.float32),
            pltpu.VMEM((CHUNK, DEG_W), jnp.float32),
            pltpu.VMEM_SHARED((N_ACC, D), jnp.float32),
            pltpu.VMEM_SHARED((N_ACC, DEG_W), jnp.float32),
            pltpu.SemaphoreType.DMA,
            pltpu.SemaphoreType.DMA,
            pltpu.SemaphoreType.DMA,
            pltpu.SemaphoreType.DMA,
        ],
        compiler_params=pltpu.CompilerParams(use_tc_tiling_on_sc=False),
    )


def _tc_body(feat_ref, s_ref, d_ref, wn_ref, ws_ref, b_ref, out_ref):
    ssum = s_ref[0]
    deg = d_ref[0, :, 0:1]
    for i in range(1, NC):
        ssum = ssum + s_ref[i]
        deg = deg + d_ref[i, :, 0:1]
    h_neigh = ssum / jnp.maximum(deg, 1.0)
    out_ref[...] = (
        jnp.dot(h_neigh, wn_ref[...], preferred_element_type=jnp.float32)
        + jnp.dot(feat_ref[...], ws_ref[...], preferred_element_type=jnp.float32)
        + b_ref[...]
    )


def _tc_finish(feat_p, ssum, dacc, wn_t, ws_t, bias2d):
    blk = 1264
    grid = N_ACC // blk
    return pl.pallas_call(
        _tc_body,
        grid=(grid,),
        in_specs=[
            pl.BlockSpec((blk, D), lambda i: (i, 0)),
            pl.BlockSpec((NC, blk, D), lambda i: (0, i, 0)),
            pl.BlockSpec((NC, blk, DEG_W), lambda i: (0, i, 0)),
            pl.BlockSpec((D, D), lambda i: (0, 0)),
            pl.BlockSpec((D, D), lambda i: (0, 0)),
            pl.BlockSpec((1, D), lambda i: (0, 0)),
        ],
        out_specs=pl.BlockSpec((blk, D), lambda i: (i, 0)),
        out_shape=jax.ShapeDtypeStruct((N_ACC, D), jnp.float32),
    )(feat_p, ssum, dacc, wn_t, ws_t, bias2d)


@jax.jit
def kernel(feat, edge_index, W_neigh, W_self, bias):
    e = edge_index.shape[1]
    cpw = -(-(-(-e // (CHUNK * NW))) // SLAB) * SLAB
    n_chunks = cpw * NW
    e_pad = n_chunks * CHUNK
    src = edge_index[0].astype(jnp.int32)
    dst = edge_index[1].astype(jnp.int32)
    pad = e_pad - e
    src_p = jnp.concatenate([src, jnp.zeros((pad,), jnp.int32)]).reshape(n_chunks, CHUNK)
    dst_p = jnp.concatenate([dst, jnp.full((pad,), N_NODES, jnp.int32)]).reshape(n_chunks, CHUNK)

    ssum, dacc = _make_sc_agg(n_chunks)(src_p, dst_p, feat)

    feat_p = jnp.zeros((N_ACC, D), feat.dtype).at[:N_NODES].set(feat)
    out = _tc_finish(feat_p, ssum, dacc, W_neigh.T, W_self.T,
                     bias.reshape(1, D))
    return out[:N_NODES]


# speedup vs baseline: 3.6079x; 3.6079x over previous
import functools

import jax
import jax.numpy as jnp
from jax import lax
from jax.experimental import pallas as pl
from jax.experimental.pallas import tpu as pltpu
from jax.experimental.pallas import tpu_sc as plsc

N_NODES = 10000
D = 128
NC, NS = 2, 16
NW = NC * NS
CHUNK = 128
N_ACC = 10112
ROWS_PER_TILE = N_ACC // NS
DEG_W = 32
SLAB = 8


def _sc_agg_body(src_hbm, dst_hbm, featbf_hbm, sum_out, deg_out,
                 src_v, dst_v, braw_a, braw_b, ones_v, zdeg_v,
                 feat_sp, accum, degacc, gsem_a, gsem_b, ssem_a, ssem_b):
    c = lax.axis_index("c")
    s = lax.axis_index("s")
    w = s * NC + c
    cpw = src_hbm.shape[0] // NW

    zero32 = jnp.zeros((32,), jnp.bfloat16)
    one32 = jnp.ones((32,), jnp.bfloat16)

    def z_rows(i, _):
        braw_a[i // 4, pl.ds((i % 4) * 32, 32)] = zero32
        return 0
    lax.fori_loop(0, CHUNK * 4, z_rows, 0)

    def z_deg(i, _):
        zdeg_v[i, pl.ds(0, 32)] = zero32
        return 0
    lax.fori_loop(0, CHUNK, z_deg, 0)

    def o_rows(i, _):
        ones_v[i, pl.ds(0, 32)] = one32
        return 0
    lax.fori_loop(0, CHUNK, o_rows, 0)

    base = s * ROWS_PER_TILE
    pltpu.sync_copy(featbf_hbm.at[pl.ds(base, ROWS_PER_TILE)],
                    feat_sp.at[pl.ds(base, ROWS_PER_TILE)])

    for k in range(ROWS_PER_TILE // CHUNK):
        pltpu.sync_copy(braw_a, accum.at[pl.ds(base + k * CHUNK, CHUNK)])
        pltpu.sync_copy(zdeg_v, degacc.at[pl.ds(base + k * CHUNK, CHUNK)])
    rem = ROWS_PER_TILE % CHUNK
    if rem:
        rbase = base + (ROWS_PER_TILE // CHUNK) * CHUNK
        pltpu.sync_copy(braw_a.at[pl.ds(0, rem)], accum.at[pl.ds(rbase, rem)])
        pltpu.sync_copy(zdeg_v.at[pl.ds(0, rem)], degacc.at[pl.ds(rbase, rem)])
    plsc.subcore_barrier()

    bufs = (braw_a, braw_b)
    gsems = (gsem_a, gsem_b)
    ssems = (ssem_a, ssem_b)

    def outer(t, _):
        cbase = w * cpw + t * SLAB
        pltpu.sync_copy(src_hbm.at[pl.ds(cbase, SLAB)], src_v)
        pltpu.sync_copy(dst_hbm.at[pl.ds(cbase, SLAB)], dst_v)
        gather = {0: pltpu.async_copy(feat_sp.at[src_v.at[0]], bufs[0], gsems[0])}
        pend = {0: [], 1: []}
        for j in range(SLAB):
            b = j % 2
            nb = 1 - b
            if j + 1 < SLAB:
                for h in pend[nb]:
                    h.wait()
                pend[nb] = []
                gather[nb] = pltpu.async_copy(
                    feat_sp.at[src_v.at[j + 1]], bufs[nb], gsems[nb])
            gather[b].wait()
            pend[b] = [
                pltpu.async_copy(bufs[b], accum.at[dst_v.at[j]], ssems[b], add=True),
                pltpu.async_copy(ones_v, degacc.at[dst_v.at[j]], ssems[b], add=True),
            ]
        for k in (0, 1):
            for h in pend[k]:
                h.wait()
        return 0
    lax.fori_loop(0, cpw // SLAB, outer, 0)

    plsc.subcore_barrier()

    pltpu.sync_copy(accum.at[pl.ds(base, ROWS_PER_TILE)],
                    sum_out.at[c, pl.ds(base, ROWS_PER_TILE)])
    pltpu.sync_copy(degacc.at[pl.ds(base, ROWS_PER_TILE)],
                    deg_out.at[c, pl.ds(base, ROWS_PER_TILE)])


def _make_sc_agg(n_chunks):
    return pl.kernel(
        _sc_agg_body,
        out_type=[
            jax.ShapeDtypeStruct((NC, N_ACC, D), jnp.bfloat16),
            jax.ShapeDtypeStruct((NC, N_ACC, DEG_W), jnp.bfloat16),
        ],
        mesh=plsc.VectorSubcoreMesh(core_axis_name="c", subcore_axis_name="s",
                                    num_cores=NC),
        scratch_types=[
            pltpu.VMEM((SLAB, CHUNK), jnp.int32),
            pltpu.VMEM((SLAB, CHUNK), jnp.int32),
            pltpu.VMEM((CHUNK, D), jnp.bfloat16),
            pltpu.VMEM((CHUNK, D), jnp.bfloat16),
            pltpu.VMEM((CHUNK, DEG_W), jnp.bfloat16),
            pltpu.VMEM((CHUNK, DEG_W), jnp.bfloat16),
            pltpu.VMEM_SHARED((N_ACC, D), jnp.bfloat16),
            pltpu.VMEM_SHARED((N_ACC, D), jnp.bfloat16),
            pltpu.VMEM_SHARED((N_ACC, DEG_W), jnp.bfloat16),
            pltpu.SemaphoreType.DMA,
            pltpu.SemaphoreType.DMA,
            pltpu.SemaphoreType.DMA,
            pltpu.SemaphoreType.DMA,
        ],
        compiler_params=pltpu.CompilerParams(use_tc_tiling_on_sc=False),
    )


def _tc_body(feat_ref, s_ref, d_ref, wn_ref, ws_ref, b_ref, out_ref):
    ssum = s_ref[0].astype(jnp.float32)
    deg = d_ref[0, :, 0:1].astype(jnp.float32)
    for i in range(1, NC):
        ssum = ssum + s_ref[i].astype(jnp.float32)
        deg = deg + d_ref[i, :, 0:1].astype(jnp.float32)
    h_neigh = ssum / jnp.maximum(deg, 1.0)
    out_ref[...] = (
        jnp.dot(h_neigh, wn_ref[...], preferred_element_type=jnp.float32)
        + jnp.dot(feat_ref[...], ws_ref[...], preferred_element_type=jnp.float32)
        + b_ref[...]
    )


def _tc_finish(feat_p, ssum, dacc, wn_t, ws_t, bias2d):
    blk = 1264
    grid = N_ACC // blk
    return pl.pallas_call(
        _tc_body,
        grid=(grid,),
        in_specs=[
            pl.BlockSpec((blk, D), lambda i: (i, 0)),
            pl.BlockSpec((NC, blk, D), lambda i: (0, i, 0)),
            pl.BlockSpec((NC, blk, DEG_W), lambda i: (0, i, 0)),
            pl.BlockSpec((D, D), lambda i: (0, 0)),
            pl.BlockSpec((D, D), lambda i: (0, 0)),
            pl.BlockSpec((1, D), lambda i: (0, 0)),
        ],
        out_specs=pl.BlockSpec((blk, D), lambda i: (i, 0)),
        out_shape=jax.ShapeDtypeStruct((N_ACC, D), jnp.float32),
    )(feat_p, ssum, dacc, wn_t, ws_t, bias2d)


@jax.jit
def kernel(feat, edge_index, W_neigh, W_self, bias):
    e = edge_index.shape[1]
    cpw = -(-(-(-e // (CHUNK * NW))) // SLAB) * SLAB
    n_chunks = cpw * NW
    e_pad = n_chunks * CHUNK
    src = edge_index[0].astype(jnp.int32)
    dst = edge_index[1].astype(jnp.int32)
    pad = e_pad - e
    src_p = jnp.concatenate([src, jnp.zeros((pad,), jnp.int32)]).reshape(n_chunks, CHUNK)
    dst_p = jnp.concatenate([dst, jnp.full((pad,), N_NODES, jnp.int32)]).reshape(n_chunks, CHUNK)

    feat_bf = jnp.zeros((N_ACC, D), jnp.bfloat16).at[:N_NODES].set(
        feat.astype(jnp.bfloat16))
    ssum, dacc = _make_sc_agg(n_chunks)(src_p, dst_p, feat_bf)

    feat_p = jnp.zeros((N_ACC, D), feat.dtype).at[:N_NODES].set(feat)
    out = _tc_finish(feat_p, ssum, dacc, W_neigh.T, W_self.T,
                     bias.reshape(1, D))
    return out[:N_NODES]


# speedup vs baseline: 3.6702x; 1.0173x over previous
import functools

import jax
import jax.numpy as jnp
from jax import lax
from jax.experimental import pallas as pl
from jax.experimental.pallas import tpu as pltpu
from jax.experimental.pallas import tpu_sc as plsc

N_NODES = 10000
D = 128
NC, NS = 2, 16
NW = NC * NS
CHUNK = 128
N_ACC = 10112
ROWS_PER_TILE = N_ACC // NS
DEG_W = 32
SLAB = 8


def _sc_agg_body(pk_hbm, featbf_hbm, sum_out, deg_out,
                 pk_v, src_a, src_b, dst_a, dst_b, braw_a, braw_b,
                 ones_v, zdeg_v,
                 feat_sp, accum, degacc, gsem_a, gsem_b, ssem_a, ssem_b):
    c = lax.axis_index("c")
    s = lax.axis_index("s")
    w = s * NC + c
    cpw = pk_hbm.shape[0] // NW

    zero32 = jnp.zeros((32,), jnp.bfloat16)
    one32 = jnp.ones((32,), jnp.bfloat16)

    def z_rows(i, _):
        braw_a[i // 4, pl.ds((i % 4) * 32, 32)] = zero32
        return 0
    lax.fori_loop(0, CHUNK * 4, z_rows, 0)

    def z_deg(i, _):
        zdeg_v[i, pl.ds(0, 32)] = zero32
        return 0
    lax.fori_loop(0, CHUNK, z_deg, 0)

    def o_rows(i, _):
        ones_v[i, pl.ds(0, 32)] = one32
        return 0
    lax.fori_loop(0, CHUNK, o_rows, 0)

    base = s * ROWS_PER_TILE
    pltpu.sync_copy(featbf_hbm.at[pl.ds(base, ROWS_PER_TILE)],
                    feat_sp.at[pl.ds(base, ROWS_PER_TILE)])

    for k in range(ROWS_PER_TILE // CHUNK):
        pltpu.sync_copy(braw_a, accum.at[pl.ds(base + k * CHUNK, CHUNK)])
        pltpu.sync_copy(zdeg_v, degacc.at[pl.ds(base + k * CHUNK, CHUNK)])
    rem = ROWS_PER_TILE % CHUNK
    if rem:
        rbase = base + (ROWS_PER_TILE // CHUNK) * CHUNK
        pltpu.sync_copy(braw_a.at[pl.ds(0, rem)], accum.at[pl.ds(rbase, rem)])
        pltpu.sync_copy(zdeg_v.at[pl.ds(0, rem)], degacc.at[pl.ds(rbase, rem)])
    plsc.subcore_barrier()

    bufs = (braw_a, braw_b)
    srcs = (src_a, src_b)
    dsts = (dst_a, dst_b)
    gsems = (gsem_a, gsem_b)
    ssems = (ssem_a, ssem_b)

    pltpu.sync_copy(pk_hbm.at[pl.ds(w * cpw, cpw)], pk_v)

    def unpack_idx(cidx, b):
        for k in range(CHUNK // 16):
            v = pk_v[cidx, pl.ds(k * 16, 16)]
            srcs[b][0, pl.ds(k * 16, 16)] = lax.bitwise_and(v, 0xFFFF)
            dsts[b][0, pl.ds(k * 16, 16)] = lax.shift_right_logical(v, 16)

    def outer(t, _):
        cbase = t * SLAB
        unpack_idx(cbase, 0)
        gather = {0: pltpu.async_copy(feat_sp.at[src_a.at[0]], bufs[0], gsems[0])}
        pend = {0: [], 1: []}
        for j in range(SLAB):
            b = j % 2
            nb = 1 - b
            if j + 1 < SLAB:
                for h in pend[nb]:
                    h.wait()
                pend[nb] = []
                unpack_idx(cbase + j + 1, nb)
                gather[nb] = pltpu.async_copy(
                    feat_sp.at[srcs[nb].at[0]], bufs[nb], gsems[nb])
            gather[b].wait()
            pend[b] = [
                pltpu.async_copy(bufs[b], accum.at[dsts[b].at[0]], ssems[b], add=True),
                pltpu.async_copy(ones_v, degacc.at[dsts[b].at[0]], ssems[b], add=True),
            ]
        for k in (0, 1):
            for h in pend[k]:
                h.wait()
        return 0
    lax.fori_loop(0, cpw // SLAB, outer, 0)

    plsc.subcore_barrier()

    pltpu.sync_copy(accum.at[pl.ds(base, ROWS_PER_TILE)],
                    sum_out.at[c, pl.ds(base, ROWS_PER_TILE)])
    pltpu.sync_copy(degacc.at[pl.ds(base, ROWS_PER_TILE)],
                    deg_out.at[c, pl.ds(base, ROWS_PER_TILE)])


def _make_sc_agg(n_chunks):
    return pl.kernel(
        _sc_agg_body,
        out_type=[
            jax.ShapeDtypeStruct((NC, N_ACC, D), jnp.bfloat16),
            jax.ShapeDtypeStruct((NC, N_ACC, DEG_W), jnp.bfloat16),
        ],
        mesh=plsc.VectorSubcoreMesh(core_axis_name="c", subcore_axis_name="s",
                                    num_cores=NC),
        scratch_types=[
            pltpu.VMEM((n_chunks // NW, CHUNK), jnp.int32),
            pltpu.VMEM((1, CHUNK), jnp.int32),
            pltpu.VMEM((1, CHUNK), jnp.int32),
            pltpu.VMEM((1, CHUNK), jnp.int32),
            pltpu.VMEM((1, CHUNK), jnp.int32),
            pltpu.VMEM((CHUNK, D), jnp.bfloat16),
            pltpu.VMEM((CHUNK, D), jnp.bfloat16),
            pltpu.VMEM((CHUNK, DEG_W), jnp.bfloat16),
            pltpu.VMEM((CHUNK, DEG_W), jnp.bfloat16),
            pltpu.VMEM_SHARED((N_ACC, D), jnp.bfloat16),
            pltpu.VMEM_SHARED((N_ACC, D), jnp.bfloat16),
            pltpu.VMEM_SHARED((N_ACC, DEG_W), jnp.bfloat16),
            pltpu.SemaphoreType.DMA,
            pltpu.SemaphoreType.DMA,
            pltpu.SemaphoreType.DMA,
            pltpu.SemaphoreType.DMA,
        ],
        compiler_params=pltpu.CompilerParams(use_tc_tiling_on_sc=False),
    )


def _tc_body(feat_ref, s_ref, d_ref, wn_ref, ws_ref, b_ref, out_ref):
    ssum = s_ref[0].astype(jnp.float32)
    deg = d_ref[0, :, 0:1].astype(jnp.float32)
    for i in range(1, NC):
        ssum = ssum + s_ref[i].astype(jnp.float32)
        deg = deg + d_ref[i, :, 0:1].astype(jnp.float32)
    h_neigh = ssum / jnp.maximum(deg, 1.0)
    out_ref[...] = (
        jnp.dot(h_neigh, wn_ref[...], preferred_element_type=jnp.float32)
        + jnp.dot(feat_ref[...], ws_ref[...], preferred_element_type=jnp.float32)
        + b_ref[...]
    )


def _tc_finish(feat_p, ssum, dacc, wn_t, ws_t, bias2d):
    blk = 1264
    grid = N_ACC // blk
    return pl.pallas_call(
        _tc_body,
        grid=(grid,),
        in_specs=[
            pl.BlockSpec((blk, D), lambda i: (i, 0)),
            pl.BlockSpec((NC, blk, D), lambda i: (0, i, 0)),
            pl.BlockSpec((NC, blk, DEG_W), lambda i: (0, i, 0)),
            pl.BlockSpec((D, D), lambda i: (0, 0)),
            pl.BlockSpec((D, D), lambda i: (0, 0)),
            pl.BlockSpec((1, D), lambda i: (0, 0)),
        ],
        out_specs=pl.BlockSpec((blk, D), lambda i: (i, 0)),
        out_shape=jax.ShapeDtypeStruct((N_ACC, D), jnp.float32),
    )(feat_p, ssum, dacc, wn_t, ws_t, bias2d)


@jax.jit
def kernel(feat, edge_index, W_neigh, W_self, bias):
    e = edge_index.shape[1]
    cpw = -(-(-(-e // (CHUNK * NW))) // SLAB) * SLAB
    n_chunks = cpw * NW
    e_pad = n_chunks * CHUNK
    src = edge_index[0].astype(jnp.int32)
    dst = edge_index[1].astype(jnp.int32)
    pad = e_pad - e
    packed = jnp.bitwise_or(src, jnp.left_shift(dst, 16))
    pk_p = jnp.concatenate(
        [packed, jnp.full((pad,), N_NODES << 16, jnp.int32)]).reshape(n_chunks, CHUNK)

    feat_bf = jnp.zeros((N_ACC, D), jnp.bfloat16).at[:N_NODES].set(
        feat.astype(jnp.bfloat16))
    ssum, dacc = _make_sc_agg(n_chunks)(pk_p, feat_bf)

    feat_p = jnp.zeros((N_ACC, D), feat.dtype).at[:N_NODES].set(feat)
    out = _tc_finish(feat_p, ssum, dacc, W_neigh.T, W_self.T,
                     bias.reshape(1, D))
    return out[:N_NODES]


# speedup vs baseline: 3.8533x; 1.0499x over previous
import functools

import jax
import jax.numpy as jnp
from jax import lax
from jax.experimental import pallas as pl
from jax.experimental.pallas import tpu as pltpu
from jax.experimental.pallas import tpu_sc as plsc

N_NODES = 10000
D = 128
NC, NS = 2, 16
NW = NC * NS
CHUNK = 128
N_ACC = 10112
ROWS_PER_TILE = N_ACC // NS
DEG_W = 32
SLAB = 8


def _sc_agg_body(pk_hbm, featbf_hbm, sum_out, deg_out,
                 pk_v, src_a, src_b, src_c, dst_a, dst_b, dst_c,
                 braw_a, braw_b, braw_c, ones_v, zdeg_v,
                 feat_sp, accum, degacc,
                 gsem_a, gsem_b, gsem_c, ssem_a, ssem_b, ssem_c):
    c = lax.axis_index("c")
    s = lax.axis_index("s")
    w = s * NC + c
    cpw = pk_hbm.shape[0] // NW

    zero32 = jnp.zeros((32,), jnp.bfloat16)
    one32 = jnp.ones((32,), jnp.bfloat16)

    def z_rows(i, _):
        braw_a[i // 4, pl.ds((i % 4) * 32, 32)] = zero32
        return 0
    lax.fori_loop(0, CHUNK * 4, z_rows, 0)

    def z_deg(i, _):
        zdeg_v[i, pl.ds(0, 32)] = zero32
        return 0
    lax.fori_loop(0, CHUNK, z_deg, 0)

    def o_rows(i, _):
        ones_v[i, pl.ds(0, 32)] = one32
        return 0
    lax.fori_loop(0, CHUNK, o_rows, 0)

    base = s * ROWS_PER_TILE
    n_feat = featbf_hbm.shape[0]
    last = ROWS_PER_TILE * NS - n_feat
    @pl.when(base + ROWS_PER_TILE <= n_feat)
    def _():
        pltpu.sync_copy(featbf_hbm.at[pl.ds(base, ROWS_PER_TILE)],
                        feat_sp.at[pl.ds(base, ROWS_PER_TILE)])
    @pl.when(base + ROWS_PER_TILE > n_feat)
    def _():
        pltpu.sync_copy(featbf_hbm.at[pl.ds(base, ROWS_PER_TILE - last)],
                        feat_sp.at[pl.ds(base, ROWS_PER_TILE - last)])

    for k in range(ROWS_PER_TILE // CHUNK):
        pltpu.sync_copy(braw_a, accum.at[pl.ds(base + k * CHUNK, CHUNK)])
        pltpu.sync_copy(zdeg_v, degacc.at[pl.ds(base + k * CHUNK, CHUNK)])
    rem = ROWS_PER_TILE % CHUNK
    if rem:
        rbase = base + (ROWS_PER_TILE // CHUNK) * CHUNK
        pltpu.sync_copy(braw_a.at[pl.ds(0, rem)], accum.at[pl.ds(rbase, rem)])
        pltpu.sync_copy(zdeg_v.at[pl.ds(0, rem)], degacc.at[pl.ds(rbase, rem)])
    plsc.subcore_barrier()

    bufs = (braw_a, braw_b, braw_c)
    srcs = (src_a, src_b, src_c)
    dsts = (dst_a, dst_b, dst_c)
    gsems = (gsem_a, gsem_b, gsem_c)
    ssems = (ssem_a, ssem_b, ssem_c)
    NB = 3

    pltpu.sync_copy(pk_hbm.at[pl.ds(w * cpw, cpw)], pk_v)

    def unpack_idx(cidx, b):
        for k in range(CHUNK // 16):
            v = pk_v[cidx, pl.ds(k * 16, 16)]
            srcs[b][0, pl.ds(k * 16, 16)] = lax.bitwise_and(v, 0xFFFF)
            dsts[b][0, pl.ds(k * 16, 16)] = lax.shift_right_logical(v, 16)

    def issue_gather(cidx, b):
        unpack_idx(cidx, b)
        return pltpu.async_copy(feat_sp.at[srcs[b].at[0]], bufs[b], gsems[b])

    def outer(t, _):
        cbase = t * SLAB
        gather = {0: issue_gather(cbase, 0), 1: issue_gather(cbase + 1, 1)}
        pend = {0: [], 1: [], 2: []}
        for j in range(SLAB):
            b = j % NB
            if j + 2 < SLAB:
                bn = (j + 2) % NB
                for h in pend[bn]:
                    h.wait()
                pend[bn] = []
                gather[bn] = issue_gather(cbase + j + 2, bn)
            gather[b].wait()
            pend[b] = [
                pltpu.async_copy(bufs[b], accum.at[dsts[b].at[0]], ssems[b], add=True),
                pltpu.async_copy(ones_v, degacc.at[dsts[b].at[0]], ssems[b], add=True),
            ]
        for k in range(NB):
            for h in pend[k]:
                h.wait()
        return 0
    lax.fori_loop(0, cpw // SLAB, outer, 0)

    plsc.subcore_barrier()

    pltpu.sync_copy(accum.at[pl.ds(base, ROWS_PER_TILE)],
                    sum_out.at[c, pl.ds(base, ROWS_PER_TILE)])
    pltpu.sync_copy(degacc.at[pl.ds(base, ROWS_PER_TILE)],
                    deg_out.at[c, pl.ds(base, ROWS_PER_TILE)])


def _make_sc_agg(n_chunks):
    return pl.kernel(
        _sc_agg_body,
        out_type=[
            jax.ShapeDtypeStruct((NC, N_ACC, D), jnp.bfloat16),
            jax.ShapeDtypeStruct((NC, N_ACC, DEG_W), jnp.bfloat16),
        ],
        mesh=plsc.VectorSubcoreMesh(core_axis_name="c", subcore_axis_name="s",
                                    num_cores=NC),
        scratch_types=[
            pltpu.VMEM((n_chunks // NW, CHUNK), jnp.int32),
            pltpu.VMEM((1, CHUNK), jnp.int32),
            pltpu.VMEM((1, CHUNK), jnp.int32),
            pltpu.VMEM((1, CHUNK), jnp.int32),
            pltpu.VMEM((1, CHUNK), jnp.int32),
            pltpu.VMEM((1, CHUNK), jnp.int32),
            pltpu.VMEM((1, CHUNK), jnp.int32),
            pltpu.VMEM((CHUNK, D), jnp.bfloat16),
            pltpu.VMEM((CHUNK, D), jnp.bfloat16),
            pltpu.VMEM((CHUNK, D), jnp.bfloat16),
            pltpu.VMEM((CHUNK, DEG_W), jnp.bfloat16),
            pltpu.VMEM((CHUNK, DEG_W), jnp.bfloat16),
            pltpu.VMEM_SHARED((N_ACC, D), jnp.bfloat16),
            pltpu.VMEM_SHARED((N_ACC, D), jnp.bfloat16),
            pltpu.VMEM_SHARED((N_ACC, DEG_W), jnp.bfloat16),
            pltpu.SemaphoreType.DMA,
            pltpu.SemaphoreType.DMA,
            pltpu.SemaphoreType.DMA,
            pltpu.SemaphoreType.DMA,
            pltpu.SemaphoreType.DMA,
            pltpu.SemaphoreType.DMA,
        ],
        compiler_params=pltpu.CompilerParams(use_tc_tiling_on_sc=False),
    )


def _tc_body(feat_ref, s_ref, d_ref, wn_ref, ws_ref, b_ref, out_ref):
    ssum = s_ref[0].astype(jnp.float32)
    deg = d_ref[0, :, 0:1].astype(jnp.float32)
    for i in range(1, NC):
        ssum = ssum + s_ref[i].astype(jnp.float32)
        deg = deg + d_ref[i, :, 0:1].astype(jnp.float32)
    h_neigh = ssum / jnp.maximum(deg, 1.0)
    out_ref[...] = (
        jnp.dot(h_neigh, wn_ref[...], preferred_element_type=jnp.float32)
        + jnp.dot(feat_ref[...], ws_ref[...], preferred_element_type=jnp.float32)
        + b_ref[...]
    )


def _tc_finish(feat, ssum, dacc, wn_t, ws_t, bias2d):
    blk = 2000
    grid = N_NODES // blk
    return pl.pallas_call(
        _tc_body,
        grid=(grid,),
        in_specs=[
            pl.BlockSpec((blk, D), lambda i: (i, 0)),
            pl.BlockSpec((NC, blk, D), lambda i: (0, i, 0)),
            pl.BlockSpec((NC, blk, DEG_W), lambda i: (0, i, 0)),
            pl.BlockSpec((D, D), lambda i: (0, 0)),
            pl.BlockSpec((D, D), lambda i: (0, 0)),
            pl.BlockSpec((1, D), lambda i: (0, 0)),
        ],
        out_specs=pl.BlockSpec((blk, D), lambda i: (i, 0)),
        out_shape=jax.ShapeDtypeStruct((N_NODES, D), jnp.float32),
    )(feat, ssum, dacc, wn_t, ws_t, bias2d)


@jax.jit
def kernel(feat, edge_index, W_neigh, W_self, bias):
    e = edge_index.shape[1]
    cpw = -(-(-(-e // (CHUNK * NW))) // SLAB) * SLAB
    n_chunks = cpw * NW
    e_pad = n_chunks * CHUNK
    src = edge_index[0].astype(jnp.int32)
    dst = edge_index[1].astype(jnp.int32)
    pad = e_pad - e
    packed = jnp.bitwise_or(src, jnp.left_shift(dst, 16))
    pk_p = jnp.concatenate(
        [packed, jnp.full((pad,), N_NODES << 16, jnp.int32)]).reshape(n_chunks, CHUNK)

    feat_bf = feat.astype(jnp.bfloat16)
    ssum, dacc = _make_sc_agg(n_chunks)(pk_p, feat_bf)

    out = _tc_finish(feat, ssum, dacc, W_neigh.T, W_self.T,
                     bias.reshape(1, D))
    return out


# speedup vs baseline: 4.0926x; 1.0621x over previous
import functools

import jax
import jax.numpy as jnp
from jax import lax
from jax.experimental import pallas as pl
from jax.experimental.pallas import tpu as pltpu
from jax.experimental.pallas import tpu_sc as plsc

N_NODES = 10000
D = 128
NC, NS = 2, 16
NW = NC * NS
N_ACC = 10048
ROWS_PER_TILE = N_ACC // NS
DEG_W = 32
SLAB = 8


def _sc_agg_body(ei_hbm, featbf_hbm, sum_out, deg_out,
                 src_sl, dst_sl, braw_a, braw_b, ones_v, zdeg_v,
                 feat_sp, accum, degacc, gsem_a, gsem_b, ssem_a, ssem_b):
    c = lax.axis_index("c")
    s = lax.axis_index("s")
    w = s * NC + c
    cpw = src_sl.shape[0]
    chunk = src_sl.shape[1]

    zero32 = jnp.zeros((32,), jnp.bfloat16)
    one32 = jnp.ones((32,), jnp.bfloat16)

    def z_rows(i, _):
        braw_a[i // 4, pl.ds((i % 4) * 32, 32)] = zero32
        return 0
    lax.fori_loop(0, chunk * 4, z_rows, 0)

    def z_deg(i, _):
        zdeg_v[i, pl.ds(0, 32)] = zero32
        return 0
    lax.fori_loop(0, chunk, z_deg, 0)

    def o_rows(i, _):
        ones_v[i, pl.ds(0, 32)] = one32
        return 0
    lax.fori_loop(0, chunk, o_rows, 0)

    base = s * ROWS_PER_TILE
    n_feat = featbf_hbm.shape[0]
    last = ROWS_PER_TILE * NS - n_feat
    @pl.when(base + ROWS_PER_TILE <= n_feat)
    def _():
        pltpu.sync_copy(featbf_hbm.at[pl.ds(base, ROWS_PER_TILE)],
                        feat_sp.at[pl.ds(base, ROWS_PER_TILE)])
    @pl.when(base + ROWS_PER_TILE > n_feat)
    def _():
        pltpu.sync_copy(featbf_hbm.at[pl.ds(base, ROWS_PER_TILE - last)],
                        feat_sp.at[pl.ds(base, ROWS_PER_TILE - last)])

    nfull = ROWS_PER_TILE // chunk
    for k in range(nfull):
        pltpu.sync_copy(braw_a, accum.at[pl.ds(base + k * chunk, chunk)])
        pltpu.sync_copy(zdeg_v, degacc.at[pl.ds(base + k * chunk, chunk)])
    rem = ROWS_PER_TILE % chunk
    if rem:
        rbase = base + nfull * chunk
        pltpu.sync_copy(braw_a.at[pl.ds(0, rem)], accum.at[pl.ds(rbase, rem)])
        pltpu.sync_copy(zdeg_v.at[pl.ds(0, rem)], degacc.at[pl.ds(rbase, rem)])

    pltpu.sync_copy(ei_hbm.at[0, pl.ds(w * cpw, cpw)], src_sl)
    pltpu.sync_copy(ei_hbm.at[1, pl.ds(w * cpw, cpw)], dst_sl)
    plsc.subcore_barrier()

    bufs = (braw_a, braw_b)
    gsems = (gsem_a, gsem_b)
    ssems = (ssem_a, ssem_b)

    def outer(t, _):
        cbase = t * SLAB
        gather = {0: pltpu.async_copy(feat_sp.at[src_sl.at[cbase]],
                                      bufs[0], gsems[0])}
        pend = {0: [], 1: []}
        for j in range(SLAB):
            b = j % 2
            nb = 1 - b
            if j + 1 < SLAB:
                for h in pend[nb]:
                    h.wait()
                pend[nb] = []
                gather[nb] = pltpu.async_copy(
                    feat_sp.at[src_sl.at[cbase + j + 1]], bufs[nb], gsems[nb])
            gather[b].wait()
            pend[b] = [
                pltpu.async_copy(bufs[b], accum.at[dst_sl.at[cbase + j]],
                                 ssems[b], add=True),
                pltpu.async_copy(ones_v, degacc.at[dst_sl.at[cbase + j]],
                                 ssems[b], add=True),
            ]
        for k in (0, 1):
            for h in pend[k]:
                h.wait()
        return 0
    lax.fori_loop(0, cpw // SLAB, outer, 0)

    plsc.subcore_barrier()

    pltpu.sync_copy(accum.at[pl.ds(base, ROWS_PER_TILE)],
                    sum_out.at[c, pl.ds(base, ROWS_PER_TILE)])
    pltpu.sync_copy(degacc.at[pl.ds(base, ROWS_PER_TILE)],
                    deg_out.at[c, pl.ds(base, ROWS_PER_TILE)])


def _make_sc_agg(n_chunks, chunk):
    return pl.kernel(
        _sc_agg_body,
        out_type=[
            jax.ShapeDtypeStruct((NC, N_ACC, D), jnp.bfloat16),
            jax.ShapeDtypeStruct((NC, N_ACC, DEG_W), jnp.bfloat16),
        ],
        mesh=plsc.VectorSubcoreMesh(core_axis_name="c", subcore_axis_name="s",
                                    num_cores=NC),
        scratch_types=[
            pltpu.VMEM((n_chunks // NW, chunk), jnp.int32),
            pltpu.VMEM((n_chunks // NW, chunk), jnp.int32),
            pltpu.VMEM((chunk, D), jnp.bfloat16),
            pltpu.VMEM((chunk, D), jnp.bfloat16),
            pltpu.VMEM((chunk, DEG_W), jnp.bfloat16),
            pltpu.VMEM((chunk, DEG_W), jnp.bfloat16),
            pltpu.VMEM_SHARED((N_ACC, D), jnp.bfloat16),
            pltpu.VMEM_SHARED((N_ACC, D), jnp.bfloat16),
            pltpu.VMEM_SHARED((N_ACC, DEG_W), jnp.bfloat16),
            pltpu.SemaphoreType.DMA,
            pltpu.SemaphoreType.DMA,
            pltpu.SemaphoreType.DMA,
            pltpu.SemaphoreType.DMA,
        ],
        compiler_params=pltpu.CompilerParams(use_tc_tiling_on_sc=False),
    )


def _tc_body(feat_ref, s_ref, d_ref, wn_ref, ws_ref, b_ref, out_ref):
    ssum = s_ref[0].astype(jnp.float32)
    deg = d_ref[0, :, 0:1].astype(jnp.float32)
    for i in range(1, NC):
        ssum = ssum + s_ref[i].astype(jnp.float32)
        deg = deg + d_ref[i, :, 0:1].astype(jnp.float32)
    h_neigh = ssum / jnp.maximum(deg, 1.0)
    out_ref[...] = (
        jnp.dot(h_neigh, wn_ref[...], preferred_element_type=jnp.float32)
        + jnp.dot(feat_ref[...], ws_ref[...], preferred_element_type=jnp.float32)
        + b_ref[...]
    )


def _tc_finish(feat, ssum, dacc, wn_t, ws_t, bias2d):
    blk = 2000
    grid = N_NODES // blk
    return pl.pallas_call(
        _tc_body,
        grid=(grid,),
        in_specs=[
            pl.BlockSpec((blk, D), lambda i: (i, 0)),
            pl.BlockSpec((NC, blk, D), lambda i: (0, i, 0)),
            pl.BlockSpec((NC, blk, DEG_W), lambda i: (0, i, 0)),
            pl.BlockSpec((D, D), lambda i: (0, 0)),
            pl.BlockSpec((D, D), lambda i: (0, 0)),
            pl.BlockSpec((1, D), lambda i: (0, 0)),
        ],
        out_specs=pl.BlockSpec((blk, D), lambda i: (i, 0)),
        out_shape=jax.ShapeDtypeStruct((N_NODES, D), jnp.float32),
    )(feat, ssum, dacc, wn_t, ws_t, bias2d)


@jax.jit
def kernel(feat, edge_index, W_neigh, W_self, bias):
    e = edge_index.shape[1]
    chunk = None
    for cand in range(128, 63, -1):
        if e % (NW * SLAB * cand) == 0:
            chunk = cand
            break
    ei = edge_index.astype(jnp.int32)
    if chunk is None:
        chunk = 128
        cpw = -(-(-(-e // (chunk * NW))) // SLAB) * SLAB
        n_chunks = cpw * NW
        pad = n_chunks * chunk - e
        ei = jnp.concatenate(
            [ei, jnp.concatenate(
                [jnp.zeros((1, pad), jnp.int32),
                 jnp.full((1, pad), N_NODES, jnp.int32)])], axis=1)
    else:
        n_chunks = e // chunk
    ei3 = ei.reshape(2, n_chunks, chunk)

    feat_bf = feat.astype(jnp.bfloat16)
    ssum, dacc = _make_sc_agg(n_chunks, chunk)(ei3, feat_bf)

    out = _tc_finish(feat, ssum, dacc, W_neigh.T, W_self.T,
                     bias.reshape(1, D))
    return out


# speedup vs baseline: 4.2594x; 1.0408x over previous
import functools

import jax
import jax.numpy as jnp
from jax import lax
from jax.experimental import pallas as pl
from jax.experimental.pallas import tpu as pltpu
from jax.experimental.pallas import tpu_sc as plsc

N_NODES = 10000
D = 128
NC, NS = 2, 16
NW = NC * NS
N_ACC = 10048
ROWS_PER_TILE = N_ACC // NS
DEG_W = 32
SLAB = 8


def _sc_agg_body(ei_hbm, featbf_hbm, sum0_out, sum1_out, deg0_out, deg1_out,
                 src_sl, dst_sl, braw_a, braw_b, ones_v, zdeg_v,
                 feat_sp, accum, degacc, gsem_a, gsem_b, ssem_a, ssem_b):
    c = lax.axis_index("c")
    s = lax.axis_index("s")
    w = s * NC + c
    cpw = src_sl.shape[0]
    chunk = src_sl.shape[1]

    zero32 = jnp.zeros((32,), jnp.bfloat16)
    one32 = jnp.ones((32,), jnp.bfloat16)

    def z_rows(i, _):
        braw_a[i // 4, pl.ds((i % 4) * 32, 32)] = zero32
        return 0
    lax.fori_loop(0, chunk * 4, z_rows, 0)

    def z_deg(i, _):
        zdeg_v[i, pl.ds(0, 32)] = zero32
        return 0
    lax.fori_loop(0, chunk, z_deg, 0)

    def o_rows(i, _):
        ones_v[i, pl.ds(0, 32)] = one32
        return 0
    lax.fori_loop(0, chunk, o_rows, 0)

    base = s * ROWS_PER_TILE
    n_feat = featbf_hbm.shape[0]
    last = ROWS_PER_TILE * NS - n_feat
    @pl.when(base + ROWS_PER_TILE <= n_feat)
    def _():
        pltpu.sync_copy(featbf_hbm.at[pl.ds(base, ROWS_PER_TILE)],
                        feat_sp.at[pl.ds(base, ROWS_PER_TILE)])
    @pl.when(base + ROWS_PER_TILE > n_feat)
    def _():
        pltpu.sync_copy(featbf_hbm.at[pl.ds(base, ROWS_PER_TILE - last)],
                        feat_sp.at[pl.ds(base, ROWS_PER_TILE - last)])

    nfull = ROWS_PER_TILE // chunk
    for k in range(nfull):
        pltpu.sync_copy(braw_a, accum.at[pl.ds(base + k * chunk, chunk)])
        pltpu.sync_copy(zdeg_v, degacc.at[pl.ds(base + k * chunk, chunk)])
    rem = ROWS_PER_TILE % chunk
    if rem:
        rbase = base + nfull * chunk
        pltpu.sync_copy(braw_a.at[pl.ds(0, rem)], accum.at[pl.ds(rbase, rem)])
        pltpu.sync_copy(zdeg_v.at[pl.ds(0, rem)], degacc.at[pl.ds(rbase, rem)])

    pltpu.sync_copy(ei_hbm.at[0, pl.ds(w * cpw, cpw)], src_sl)
    pltpu.sync_copy(ei_hbm.at[1, pl.ds(w * cpw, cpw)], dst_sl)
    plsc.subcore_barrier()

    bufs = (braw_a, braw_b)
    gsems = (gsem_a, gsem_b)
    ssems = (ssem_a, ssem_b)

    def outer(t, _):
        cbase = t * SLAB
        gather = {0: pltpu.async_copy(feat_sp.at[src_sl.at[cbase]],
                                      bufs[0], gsems[0])}
        pend = {0: [], 1: []}
        for j in range(SLAB):
            b = j % 2
            nb = 1 - b
            if j + 1 < SLAB:
                for h in pend[nb]:
                    h.wait()
                pend[nb] = []
                gather[nb] = pltpu.async_copy(
                    feat_sp.at[src_sl.at[cbase + j + 1]], bufs[nb], gsems[nb])
            gather[b].wait()
            pend[b] = [
                pltpu.async_copy(bufs[b], accum.at[dst_sl.at[cbase + j]],
                                 ssems[b], add=True),
                pltpu.async_copy(ones_v, degacc.at[dst_sl.at[cbase + j]],
                                 ssems[b], add=True),
            ]
        for k in (0, 1):
            for h in pend[k]:
                h.wait()
        return 0
    lax.fori_loop(0, cpw // SLAB, outer, 0)

    plsc.subcore_barrier()

    @pl.when(c == 0)
    def _():
        pltpu.sync_copy(accum.at[pl.ds(base, ROWS_PER_TILE)],
                        sum0_out.at[pl.ds(base, ROWS_PER_TILE)])
        pltpu.sync_copy(degacc.at[pl.ds(base, ROWS_PER_TILE)],
                        deg0_out.at[pl.ds(base, ROWS_PER_TILE)])
    @pl.when(c == 1)
    def _():
        pltpu.sync_copy(accum.at[pl.ds(base, ROWS_PER_TILE)],
                        sum1_out.at[pl.ds(base, ROWS_PER_TILE)])
        pltpu.sync_copy(degacc.at[pl.ds(base, ROWS_PER_TILE)],
                        deg1_out.at[pl.ds(base, ROWS_PER_TILE)])


def _make_sc_agg(n_chunks, chunk):
    return pl.kernel(
        _sc_agg_body,
        out_type=[
            jax.ShapeDtypeStruct((N_ACC, D), jnp.bfloat16),
            jax.ShapeDtypeStruct((N_ACC, D), jnp.bfloat16),
            jax.ShapeDtypeStruct((N_ACC, DEG_W), jnp.bfloat16),
            jax.ShapeDtypeStruct((N_ACC, DEG_W), jnp.bfloat16),
        ],
        mesh=plsc.VectorSubcoreMesh(core_axis_name="c", subcore_axis_name="s",
                                    num_cores=NC),
        scratch_types=[
            pltpu.VMEM((n_chunks // NW, chunk), jnp.int32),
            pltpu.VMEM((n_chunks // NW, chunk), jnp.int32),
            pltpu.VMEM((chunk, D), jnp.bfloat16),
            pltpu.VMEM((chunk, D), jnp.bfloat16),
            pltpu.VMEM((chunk, DEG_W), jnp.bfloat16),
            pltpu.VMEM((chunk, DEG_W), jnp.bfloat16),
            pltpu.VMEM_SHARED((N_ACC, D), jnp.bfloat16),
            pltpu.VMEM_SHARED((N_ACC, D), jnp.bfloat16),
            pltpu.VMEM_SHARED((N_ACC, DEG_W), jnp.bfloat16),
            pltpu.SemaphoreType.DMA,
            pltpu.SemaphoreType.DMA,
            pltpu.SemaphoreType.DMA,
            pltpu.SemaphoreType.DMA,
        ],
        compiler_params=pltpu.CompilerParams(use_tc_tiling_on_sc=False),
    )


def _tc_body(feat_ref, s0_ref, s1_ref, d0_ref, d1_ref,
             wn_ref, ws_ref, b_ref, out_ref):
    ssum = s0_ref[...].astype(jnp.float32) + s1_ref[...].astype(jnp.float32)
    deg = (d0_ref[:, 0:1].astype(jnp.float32)
           + d1_ref[:, 0:1].astype(jnp.float32))
    h_neigh = ssum / jnp.maximum(deg, 1.0)
    out_ref[...] = (
        jnp.dot(h_neigh, wn_ref[...], preferred_element_type=jnp.float32)
        + jnp.dot(feat_ref[...], ws_ref[...], preferred_element_type=jnp.float32)
        + b_ref[...]
    )


def _tc_finish(feat, s0, s1, d0, d1, wn_t, ws_t, bias2d):
    blk = 2000
    grid = N_NODES // blk
    return pl.pallas_call(
        _tc_body,
        grid=(grid,),
        in_specs=[
            pl.BlockSpec((blk, D), lambda i: (i, 0)),
            pl.BlockSpec((blk, D), lambda i: (i, 0)),
            pl.BlockSpec((blk, D), lambda i: (i, 0)),
            pl.BlockSpec((blk, DEG_W), lambda i: (i, 0)),
            pl.BlockSpec((blk, DEG_W), lambda i: (i, 0)),
            pl.BlockSpec((D, D), lambda i: (0, 0)),
            pl.BlockSpec((D, D), lambda i: (0, 0)),
            pl.BlockSpec((1, D), lambda i: (0, 0)),
        ],
        out_specs=pl.BlockSpec((blk, D), lambda i: (i, 0)),
        out_shape=jax.ShapeDtypeStruct((N_NODES, D), jnp.float32),
    )(feat, s0, s1, d0, d1, wn_t, ws_t, bias2d)


@jax.jit
def kernel(feat, edge_index, W_neigh, W_self, bias):
    e = edge_index.shape[1]
    chunk = None
    for cand in range(128, 63, -1):
        if e % (NW * SLAB * cand) == 0:
            chunk = cand
            break
    ei = edge_index.astype(jnp.int32)
    if chunk is None:
        chunk = 128
        cpw = -(-(-(-e // (chunk * NW))) // SLAB) * SLAB
        n_chunks = cpw * NW
        pad = n_chunks * chunk - e
        ei = jnp.concatenate(
            [ei, jnp.concatenate(
                [jnp.zeros((1, pad), jnp.int32),
                 jnp.full((1, pad), N_NODES, jnp.int32)])], axis=1)
    else:
        n_chunks = e // chunk
    ei3 = ei.reshape(2, n_chunks, chunk)

    feat_bf = feat.astype(jnp.bfloat16)
    s0, s1, d0, d1 = _make_sc_agg(n_chunks, chunk)(ei3, feat_bf)

    out = _tc_finish(feat, s0, s1, d0, d1, W_neigh.T, W_self.T,
                     bias.reshape(1, D))
    return out


# speedup vs baseline: 4.3566x; 1.0228x over previous
import functools

import jax
import jax.numpy as jnp
from jax import lax
from jax.experimental import pallas as pl
from jax.experimental.pallas import tpu as pltpu
from jax.experimental.pallas import tpu_sc as plsc

N_NODES = 10000
D = 128
NC, NS = 2, 16
NW = NC * NS
N_ACC = 10048
ROWS_PER_TILE = N_ACC // NS
DEG_W = 32
SLAB = 16


def _sc_agg_body(ei_hbm, featbf_hbm, sum0_out, sum1_out, deg0_out, deg1_out,
                 src_sl, dst_sl, braw_a, braw_b, ones_v, zdeg_v,
                 feat_sp, accum, degacc, gsem_a, gsem_b, ssem_a, ssem_b):
    c = lax.axis_index("c")
    s = lax.axis_index("s")
    w = s * NC + c
    cpw = src_sl.shape[0]
    chunk = src_sl.shape[1]

    zero32 = jnp.zeros((32,), jnp.bfloat16)
    one32 = jnp.ones((32,), jnp.bfloat16)

    def z_rows(i, _):
        braw_a[i // 4, pl.ds((i % 4) * 32, 32)] = zero32
        return 0
    lax.fori_loop(0, chunk * 4, z_rows, 0)

    def z_deg(i, _):
        zdeg_v[i, pl.ds(0, 32)] = zero32
        return 0
    lax.fori_loop(0, chunk, z_deg, 0)

    def o_rows(i, _):
        ones_v[i, pl.ds(0, 32)] = one32
        return 0
    lax.fori_loop(0, chunk, o_rows, 0)

    base = s * ROWS_PER_TILE
    n_feat = featbf_hbm.shape[0]
    last = ROWS_PER_TILE * NS - n_feat
    @pl.when(base + ROWS_PER_TILE <= n_feat)
    def _():
        pltpu.sync_copy(featbf_hbm.at[pl.ds(base, ROWS_PER_TILE)],
                        feat_sp.at[pl.ds(base, ROWS_PER_TILE)])
    @pl.when(base + ROWS_PER_TILE > n_feat)
    def _():
        pltpu.sync_copy(featbf_hbm.at[pl.ds(base, ROWS_PER_TILE - last)],
                        feat_sp.at[pl.ds(base, ROWS_PER_TILE - last)])

    nfull = ROWS_PER_TILE // chunk
    for k in range(nfull):
        pltpu.sync_copy(braw_a, accum.at[pl.ds(base + k * chunk, chunk)])
        pltpu.sync_copy(zdeg_v, degacc.at[pl.ds(base + k * chunk, chunk)])
    rem = ROWS_PER_TILE % chunk
    if rem:
        rbase = base + nfull * chunk
        pltpu.sync_copy(braw_a.at[pl.ds(0, rem)], accum.at[pl.ds(rbase, rem)])
        pltpu.sync_copy(zdeg_v.at[pl.ds(0, rem)], degacc.at[pl.ds(rbase, rem)])

    pltpu.sync_copy(ei_hbm.at[0, pl.ds(w * cpw, cpw)], src_sl)
    pltpu.sync_copy(ei_hbm.at[1, pl.ds(w * cpw, cpw)], dst_sl)
    plsc.subcore_barrier()

    bufs = (braw_a, braw_b)
    gsems = (gsem_a, gsem_b)
    ssems = (ssem_a, ssem_b)

    def outer(t, _):
        cbase = t * SLAB
        gather = {0: pltpu.async_copy(feat_sp.at[src_sl.at[cbase]],
                                      bufs[0], gsems[0])}
        pend = {0: [], 1: []}
        for j in range(SLAB):
            b = j % 2
            nb = 1 - b
            if j + 1 < SLAB:
                for h in pend[nb]:
                    h.wait()
                pend[nb] = []
                gather[nb] = pltpu.async_copy(
                    feat_sp.at[src_sl.at[cbase + j + 1]], bufs[nb], gsems[nb])
            gather[b].wait()
            pend[b] = [
                pltpu.async_copy(bufs[b], accum.at[dst_sl.at[cbase + j]],
                                 ssems[b], add=True),
                pltpu.async_copy(ones_v, degacc.at[dst_sl.at[cbase + j]],
                                 ssems[b], add=True),
            ]
        for k in (0, 1):
            for h in pend[k]:
                h.wait()
        return 0
    lax.fori_loop(0, cpw // SLAB, outer, 0)

    plsc.subcore_barrier()

    @pl.when(c == 0)
    def _():
        pltpu.sync_copy(accum.at[pl.ds(base, ROWS_PER_TILE)],
                        sum0_out.at[pl.ds(base, ROWS_PER_TILE)])
        pltpu.sync_copy(degacc.at[pl.ds(base, ROWS_PER_TILE)],
                        deg0_out.at[pl.ds(base, ROWS_PER_TILE)])
    @pl.when(c == 1)
    def _():
        pltpu.sync_copy(accum.at[pl.ds(base, ROWS_PER_TILE)],
                        sum1_out.at[pl.ds(base, ROWS_PER_TILE)])
        pltpu.sync_copy(degacc.at[pl.ds(base, ROWS_PER_TILE)],
                        deg1_out.at[pl.ds(base, ROWS_PER_TILE)])


def _make_sc_agg(n_chunks, chunk):
    return pl.kernel(
        _sc_agg_body,
        out_type=[
            jax.ShapeDtypeStruct((N_ACC, D), jnp.bfloat16),
            jax.ShapeDtypeStruct((N_ACC, D), jnp.bfloat16),
            jax.ShapeDtypeStruct((N_ACC, DEG_W), jnp.bfloat16),
            jax.ShapeDtypeStruct((N_ACC, DEG_W), jnp.bfloat16),
        ],
        mesh=plsc.VectorSubcoreMesh(core_axis_name="c", subcore_axis_name="s",
                                    num_cores=NC),
        scratch_types=[
            pltpu.VMEM((n_chunks // NW, chunk), jnp.int32),
            pltpu.VMEM((n_chunks // NW, chunk), jnp.int32),
            pltpu.VMEM((chunk, D), jnp.bfloat16),
            pltpu.VMEM((chunk, D), jnp.bfloat16),
            pltpu.VMEM((chunk, DEG_W), jnp.bfloat16),
            pltpu.VMEM((chunk, DEG_W), jnp.bfloat16),
            pltpu.VMEM_SHARED((N_ACC, D), jnp.bfloat16),
            pltpu.VMEM_SHARED((N_ACC, D), jnp.bfloat16),
            pltpu.VMEM_SHARED((N_ACC, DEG_W), jnp.bfloat16),
            pltpu.SemaphoreType.DMA,
            pltpu.SemaphoreType.DMA,
            pltpu.SemaphoreType.DMA,
            pltpu.SemaphoreType.DMA,
        ],
        compiler_params=pltpu.CompilerParams(use_tc_tiling_on_sc=False),
    )


def _tc_body(feat_ref, s0_ref, s1_ref, d0_ref, d1_ref,
             wn_ref, ws_ref, b_ref, out_ref):
    ssum = s0_ref[...].astype(jnp.float32) + s1_ref[...].astype(jnp.float32)
    deg = (d0_ref[:, 0:1].astype(jnp.float32)
           + d1_ref[:, 0:1].astype(jnp.float32))
    h_neigh = ssum / jnp.maximum(deg, 1.0)
    out_ref[...] = (
        jnp.dot(h_neigh, wn_ref[...], preferred_element_type=jnp.float32)
        + jnp.dot(feat_ref[...], ws_ref[...], preferred_element_type=jnp.float32)
        + b_ref[...]
    )


def _tc_finish(feat, s0, s1, d0, d1, wn_t, ws_t, bias2d):
    blk = 2000
    grid = N_NODES // blk
    return pl.pallas_call(
        _tc_body,
        grid=(grid,),
        in_specs=[
            pl.BlockSpec((blk, D), lambda i: (i, 0)),
            pl.BlockSpec((blk, D), lambda i: (i, 0)),
            pl.BlockSpec((blk, D), lambda i: (i, 0)),
            pl.BlockSpec((blk, DEG_W), lambda i: (i, 0)),
            pl.BlockSpec((blk, DEG_W), lambda i: (i, 0)),
            pl.BlockSpec((D, D), lambda i: (0, 0)),
            pl.BlockSpec((D, D), lambda i: (0, 0)),
            pl.BlockSpec((1, D), lambda i: (0, 0)),
        ],
        out_specs=pl.BlockSpec((blk, D), lambda i: (i, 0)),
        out_shape=jax.ShapeDtypeStruct((N_NODES, D), jnp.float32),
    )(feat, s0, s1, d0, d1, wn_t, ws_t, bias2d)


@jax.jit
def kernel(feat, edge_index, W_neigh, W_self, bias):
    e = edge_index.shape[1]
    chunk = None
    for cand in range(128, 63, -1):
        if e % (NW * SLAB * cand) == 0:
            chunk = cand
            break
    ei = edge_index.astype(jnp.int32)
    if chunk is None:
        chunk = 128
        cpw = -(-(-(-e // (chunk * NW))) // SLAB) * SLAB
        n_chunks = cpw * NW
        pad = n_chunks * chunk - e
        ei = jnp.concatenate(
            [ei, jnp.concatenate(
                [jnp.zeros((1, pad), jnp.int32),
                 jnp.full((1, pad), N_NODES, jnp.int32)])], axis=1)
    else:
        n_chunks = e // chunk
    ei3 = ei.reshape(2, n_chunks, chunk)

    feat_bf = feat.astype(jnp.bfloat16)
    s0, s1, d0, d1 = _make_sc_agg(n_chunks, chunk)(ei3, feat_bf)

    out = _tc_finish(feat, s0, s1, d0, d1, W_neigh.T, W_self.T,
                     bias.reshape(1, D))
    return out
